# Initial kernel scaffold; baseline (speedup 1.0000x reference)
import jax
import jax.numpy as jnp
from jax.experimental import pallas as pl


def kernel(pos_pair, neg_pair, emb):
    raise NotImplementedError("write your pallas kernel here")


# speedup vs baseline: 1.7999x; 1.7999x over previous
import functools

import jax
import jax.numpy as jnp
from jax import lax
from jax.experimental import pallas as pl
from jax.experimental.pallas import tpu as pltpu
from jax.experimental.pallas import tpu_sc as plsc

NC = 2
NS = 16
NW = NC * NS
LANES = 16
CHUNK = 128


def _sc_body(idx_hbm, emb_hbm, out_hbm,
             idx_o0, idx_d0, idx_o1, idx_d1,
             rows_o0, rows_d0, rows_o1, rows_d1,
             acc_v, sem0, sem1):
  B = idx_hbm.shape[1]
  per_w = B // NW
  n_chunks = per_w // CHUNK

  wid = lax.axis_index("s") * NC + lax.axis_index("c")
  base = wid * per_w

  idx_bufs = [(idx_o0, idx_d0), (idx_o1, idx_d1)]
  row_bufs = [(rows_o0, rows_d0), (rows_o1, rows_d1)]
  sems = [sem0, sem1]

  steps = [(g, c) for g in range(2) for c in range(n_chunks)]

  def start(s):
    g, c = steps[s]
    b = s % 2
    io, id_ = idx_bufs[b]
    ro, rd = row_bufs[b]
    off = base + c * CHUNK
    pltpu.sync_copy(idx_hbm.at[2 * g, pl.ds(off, CHUNK)], io)
    pltpu.sync_copy(idx_hbm.at[2 * g + 1, pl.ds(off, CHUNK)], id_)
    ho = pltpu.async_copy(emb_hbm.at[io], ro, sems[b])
    hd = pltpu.async_copy(emb_hbm.at[id_], rd, sems[b])
    return (ho, hd)

  inflight = {0: start(0)}

  zero = jnp.zeros((LANES,), jnp.float32)
  for g in range(2):
    accs = tuple(zero for _ in range(24))
    for c in range(n_chunks):
      s = g * n_chunks + c
      if s + 1 < len(steps):
        inflight[s + 1] = start(s + 1)
      ho, hd = inflight.pop(s)
      ho.wait()
      hd.wait()
      b = s % 2
      ro, rd = row_bufs[b]

      def body(i, carry, ro=ro, rd=rd):
        a = list(carry)
        for j in range(8):
          o = ro[i, pl.ds(j * LANES, LANES)]
          d = rd[i, pl.ds(j * LANES, LANES)]
          a[3 * j + 0] = a[3 * j + 0] + o * d
          a[3 * j + 1] = a[3 * j + 1] + o * o
          a[3 * j + 2] = a[3 * j + 2] + d * d
        return tuple(a)

      accs = lax.fori_loop(0, CHUNK, body, accs)

    for j in range(8):
      for k in range(3):
        acc_v[3 * g + k, pl.ds(j * LANES, LANES)] = accs[3 * j + k]

  pltpu.sync_copy(acc_v, out_hbm.at[wid])


def kernel(pos_pair, neg_pair, emb):
  B = pos_pair.shape[0]
  idx = jnp.stack([pos_pair[:, 0], pos_pair[:, 1],
                   neg_pair[:, 0], neg_pair[:, 1]]).astype(jnp.int32)

  mesh = plsc.VectorSubcoreMesh(core_axis_name="c", subcore_axis_name="s",
                                num_cores=NC, num_subcores=NS)
  partials = pl.kernel(
      _sc_body,
      out_type=jax.ShapeDtypeStruct((NW, 6, 128), jnp.float32),
      mesh=mesh,
      scratch_types=[
          pltpu.VMEM((CHUNK,), jnp.int32),
          pltpu.VMEM((CHUNK,), jnp.int32),
          pltpu.VMEM((CHUNK,), jnp.int32),
          pltpu.VMEM((CHUNK,), jnp.int32),
          pltpu.VMEM((CHUNK, 128), jnp.float32),
          pltpu.VMEM((CHUNK, 128), jnp.float32),
          pltpu.VMEM((CHUNK, 128), jnp.float32),
          pltpu.VMEM((CHUNK, 128), jnp.float32),
          pltpu.VMEM((6, 128), jnp.float32),
          pltpu.SemaphoreType.DMA,
          pltpu.SemaphoreType.DMA,
      ],
  )(idx, emb)

  p = jnp.sum(partials, axis=0)
  eps = jnp.float32(1e-8)

  def cos(num, so, sd):
    return num / (jnp.maximum(jnp.sqrt(so), eps) * jnp.maximum(jnp.sqrt(sd), eps))

  pos_dist = cos(p[0], p[1], p[2])
  neg_dist = cos(p[3], p[4], p[5])
  return (pos_dist, neg_dist)


# speedup vs baseline: 1.9948x; 1.1083x over previous
import jax
import jax.numpy as jnp
from jax import lax
from jax.experimental import pallas as pl
from jax.experimental.pallas import tpu as pltpu
from jax.experimental.pallas import tpu_sc as plsc

NC = 2
NS = 16
NW = NC * NS
LANES = 16
CHUNK = 128
NBUF = 3


def _sc_body(idx_hbm, emb_hbm, out_hbm,
             idx_v,
             ro0, rd0, ro1, rd1, ro2, rd2,
             acc_v, sem0, sem1, sem2):
  n_chunks = idx_hbm.shape[2]

  wid = lax.axis_index("s") * NC + lax.axis_index("c")

  row_bufs = [(ro0, rd0), (ro1, rd1), (ro2, rd2)]
  sems = [sem0, sem1, sem2]

  pltpu.sync_copy(idx_hbm.at[wid], idx_v)

  steps = [(g, c) for g in range(2) for c in range(n_chunks)]

  def start(s):
    g, c = steps[s]
    b = s % NBUF
    ro, rd = row_bufs[b]
    ho = pltpu.async_copy(emb_hbm.at[idx_v.at[2 * g, c]], ro, sems[b])
    hd = pltpu.async_copy(emb_hbm.at[idx_v.at[2 * g + 1, c]], rd, sems[b])
    return (ho, hd)

  inflight = {s: start(s) for s in range(min(NBUF - 1, len(steps)))}

  zero = jnp.zeros((LANES,), jnp.float32)
  for g in range(2):
    accs = tuple(zero for _ in range(24))
    for c in range(n_chunks):
      s = g * n_chunks + c
      nxt = s + NBUF - 1
      if nxt < len(steps):
        inflight[nxt] = start(nxt)
      ho, hd = inflight.pop(s)
      ho.wait()
      hd.wait()
      ro, rd = row_bufs[s % NBUF]

      def body(i2, carry, ro=ro, rd=rd):
        a = list(carry)
        for u in range(2):
          i = 2 * i2 + u
          for j in range(8):
            o = ro[i, pl.ds(j * LANES, LANES)]
            d = rd[i, pl.ds(j * LANES, LANES)]
            a[3 * j + 0] = a[3 * j + 0] + o * d
            a[3 * j + 1] = a[3 * j + 1] + o * o
            a[3 * j + 2] = a[3 * j + 2] + d * d
        return tuple(a)

      accs = lax.fori_loop(0, CHUNK // 2, body, accs)

    for j in range(8):
      for k in range(3):
        acc_v[3 * g + k, pl.ds(j * LANES, LANES)] = accs[3 * j + k]

  pltpu.sync_copy(acc_v, out_hbm.at[wid])


def kernel(pos_pair, neg_pair, emb):
  B = pos_pair.shape[0]
  per_w = B // NW
  n_chunks = per_w // CHUNK
  idx = jnp.stack([pos_pair[:, 0], pos_pair[:, 1],
                   neg_pair[:, 0], neg_pair[:, 1]]).astype(jnp.int32)
  idx = idx.reshape(4, NW, n_chunks, CHUNK).transpose(1, 0, 2, 3)

  mesh = plsc.VectorSubcoreMesh(core_axis_name="c", subcore_axis_name="s",
                                num_cores=NC, num_subcores=NS)
  partials = pl.kernel(
      _sc_body,
      out_type=jax.ShapeDtypeStruct((NW, 6, 128), jnp.float32),
      mesh=mesh,
      scratch_types=[
          pltpu.VMEM((4, n_chunks, CHUNK), jnp.int32),
          pltpu.VMEM((CHUNK, 128), jnp.float32),
          pltpu.VMEM((CHUNK, 128), jnp.float32),
          pltpu.VMEM((CHUNK, 128), jnp.float32),
          pltpu.VMEM((CHUNK, 128), jnp.float32),
          pltpu.VMEM((CHUNK, 128), jnp.float32),
          pltpu.VMEM((CHUNK, 128), jnp.float32),
          pltpu.VMEM((6, 128), jnp.float32),
          pltpu.SemaphoreType.DMA,
          pltpu.SemaphoreType.DMA,
          pltpu.SemaphoreType.DMA,
      ],
  )(idx, emb)

  p = jnp.sum(partials, axis=0)
  eps = jnp.float32(1e-8)

  def cos(num, so, sd):
    return num / (jnp.maximum(jnp.sqrt(so), eps) * jnp.maximum(jnp.sqrt(sd), eps))

  pos_dist = cos(p[0], p[1], p[2])
  neg_dist = cos(p[3], p[4], p[5])
  return (pos_dist, neg_dist)
